# baseline (speedup 1.0000x reference)
import functools

import jax
import jax.numpy as jnp
from jax import lax
from jax.experimental import pallas as pl
from jax.experimental.pallas import tpu as pltpu
from jax.experimental.pallas import tpu_sc as plsc

_NC = 2
_NS = 16
_NW = _NC * _NS
_NBUF = 4


def _make_gather(num_rows, nt):
    mesh = plsc.VectorSubcoreMesh(core_axis_name="c", subcore_axis_name="s")

    @functools.partial(
        pl.kernel,
        out_type=jax.ShapeDtypeStruct((nt, 8, _NW, 8, 128), jnp.float32),
        mesh=mesh,
        scratch_types=[
            pltpu.VMEM((nt, 128), jnp.int32),
            pltpu.VMEM((_NBUF, 128, 128), jnp.float32),
            pltpu.VMEM((_NBUF, 8, 8, 128), jnp.float32),
            [pltpu.SemaphoreType.DMA] * _NBUF,
            [pltpu.SemaphoreType.DMA] * _NBUF,
        ],
        compiler_params=pltpu.CompilerParams(use_tc_tiling_on_sc=False,
                                             needs_layout_passes=False),
    )
    def gather(idx_hbm, table_hbm, out_hbm, idx_v, rows_v, tout_v, gsems, osems):
        wid = lax.axis_index("s") * _NC + lax.axis_index("c")
        pltpu.sync_copy(idx_hbm.at[wid], idx_v)

        iota = lax.iota(jnp.int32, 16)
        dv = [16 * k + iota for k in range(4)]
        dhi = [v >> 3 for v in dv]
        dlo = [v & 7 for v in dv]

        for b in range(_NBUF):
            pltpu.async_copy(table_hbm.at[0].at[idx_v.at[b]], rows_v.at[b],
                             gsems[b])

        def transpose_chunk(b):
            rows = rows_v.at[b]
            tout = tout_v.at[b]

            @pl.loop(0, 128, unroll=8)
            def _(bl0):
                blv = (jnp.full((16,), bl0, jnp.int32) + iota) & 127
                for k in range(4):
                    vals = plsc.load_gather(rows, [blv, dv[k]])
                    plsc.store_scatter(tout, [dhi[k], dlo[k], blv], vals)

        def chunk(t, b, osem_wait, refire):
            pltpu.make_async_copy(table_hbm.at[0].at[idx_v.at[t]], rows_v.at[b],
                                  gsems[b]).wait()
            if osem_wait:
                pltpu.make_async_copy(out_hbm.at[0, :, 0], tout_v.at[b],
                                      osems[b]).wait()
            transpose_chunk(b)
            if refire:
                pltpu.async_copy(table_hbm.at[0].at[idx_v.at[t + _NBUF]],
                                 rows_v.at[b], gsems[b])
            pltpu.async_copy(tout_v.at[b], out_hbm.at[t, :, wid], osems[b])

        for b in range(_NBUF):
            chunk(b, b, osem_wait=False, refire=True)

        nsteps = nt // _NBUF

        @pl.loop(1, nsteps - 1)
        def _(s):
            for b in range(_NBUF):
                chunk(s * _NBUF + b, b, osem_wait=True, refire=True)

        for b in range(_NBUF):
            chunk((nsteps - 1) * _NBUF + b, b, osem_wait=True, refire=False)
        for b in range(_NBUF):
            pltpu.make_async_copy(out_hbm.at[0, :, 0], tout_v.at[b],
                                  osems[b]).wait()

    return gather


def kernel(text, seq_len, text_embed_weight):
    bsz, nt = text.shape
    num_rows, dim = text_embed_weight.shape
    idx3 = text.astype(jnp.int32).reshape(_NW, 128, nt).transpose(0, 2, 1)
    w3 = jnp.pad(text_embed_weight, ((0, 0), (0, 128 - dim))).reshape(
        1, num_rows, 128)
    gather = _make_gather(num_rows, nt)
    out5 = gather(idx3, w3)
    return out5.transpose(2, 4, 0, 1, 3).reshape(bsz, nt, dim)


# speedup vs baseline: 1.2407x; 1.2407x over previous
import functools

import jax
import jax.numpy as jnp
from jax import lax
from jax.experimental import pallas as pl
from jax.experimental.pallas import tpu as pltpu
from jax.experimental.pallas import tpu_sc as plsc

_NC = 2
_NS = 16
_NW = _NC * _NS
_NBUF = 4


def _make_gather(num_rows, nt):
    mesh = plsc.VectorSubcoreMesh(core_axis_name="c", subcore_axis_name="s")

    @functools.partial(
        pl.kernel,
        out_type=jax.ShapeDtypeStruct((nt, 8, _NW, 8, 128), jnp.float32),
        mesh=mesh,
        scratch_types=[
            pltpu.VMEM((nt, 128), jnp.int32),
            pltpu.VMEM((_NBUF, 128, 128), jnp.float32),
            pltpu.VMEM((_NBUF, 8, 8, 128), jnp.float32),
            [pltpu.SemaphoreType.DMA] * _NBUF,
            [pltpu.SemaphoreType.DMA] * _NBUF,
        ],
        compiler_params=pltpu.CompilerParams(use_tc_tiling_on_sc=False,
                                             needs_layout_passes=False),
    )
    def gather(idx_hbm, table_hbm, out_hbm, idx_v, rows_v, tout_v, gsems, osems):
        wid = lax.axis_index("s") * _NC + lax.axis_index("c")
        pltpu.sync_copy(idx_hbm.at[wid], idx_v)

        iota = lax.iota(jnp.int32, 16)
        dv = [16 * k + iota for k in range(4)]
        dhi = [v >> 3 for v in dv]
        dlo = [v & 7 for v in dv]

        for b in range(_NBUF):
            pltpu.async_copy(table_hbm.at[0].at[idx_v.at[b]], rows_v.at[b],
                             gsems[b])

        def transpose_chunk(b):
            rows = rows_v.at[b]
            tout = tout_v.at[b]

            @plsc.parallel_loop(0, 128, unroll=8)
            def _(bl0):
                blv = (jnp.full((16,), bl0, jnp.int32) + iota) & 127
                for k in range(4):
                    vals = plsc.load_gather(rows, [blv, dv[k]])
                    plsc.store_scatter(tout, [dhi[k], dlo[k], blv], vals)

        def chunk(t, b, osem_wait, refire):
            pltpu.make_async_copy(table_hbm.at[0].at[idx_v.at[t]], rows_v.at[b],
                                  gsems[b]).wait()
            if osem_wait:
                pltpu.make_async_copy(out_hbm.at[0, :, 0], tout_v.at[b],
                                      osems[b]).wait()
            transpose_chunk(b)
            if refire:
                pltpu.async_copy(table_hbm.at[0].at[idx_v.at[t + _NBUF]],
                                 rows_v.at[b], gsems[b])
            pltpu.async_copy(tout_v.at[b], out_hbm.at[t, :, wid], osems[b])

        for b in range(_NBUF):
            chunk(b, b, osem_wait=False, refire=True)

        nsteps = nt // _NBUF

        @pl.loop(1, nsteps - 1)
        def _(s):
            for b in range(_NBUF):
                chunk(s * _NBUF + b, b, osem_wait=True, refire=True)

        for b in range(_NBUF):
            chunk((nsteps - 1) * _NBUF + b, b, osem_wait=True, refire=False)
        for b in range(_NBUF):
            pltpu.make_async_copy(out_hbm.at[0, :, 0], tout_v.at[b],
                                  osems[b]).wait()

    return gather


def kernel(text, seq_len, text_embed_weight):
    bsz, nt = text.shape
    num_rows, dim = text_embed_weight.shape
    idx3 = text.astype(jnp.int32).reshape(_NW, 128, nt).transpose(0, 2, 1)
    w3 = jnp.pad(text_embed_weight, ((0, 0), (0, 128 - dim))).reshape(
        1, num_rows, 128)
    gather = _make_gather(num_rows, nt)
    out5 = gather(idx3, w3)
    return out5.transpose(2, 4, 0, 1, 3).reshape(bsz, nt, dim)
